# baseline (speedup 1.0000x reference)
import jax
import jax.numpy as jnp
from jax import lax
from jax.experimental import pallas as pl
from jax.experimental.pallas import tpu as pltpu
from jax.experimental.pallas import tpu_sc as plsc

_B = 16384
_D = 64
_K = 20
_S = _K + 1
_NC = 2
_NS = 16
_NW = _NC * _NS
_CHUNK = _B // _NW
_NSTEP = 8
_BS = _CHUNK // _NSTEP
_LANES = 16
_DV = _D // _LANES
_NG = _BS // _LANES


def _sc_scores_body(cidx_hbm, kidx_hbm, in_hbm, out_hbm, scores_hbm,
                    cidx_v, kidx_v, crow_v, orow_v, scores_v, sem):
    wid = lax.axis_index("s") * _NC + lax.axis_index("c")

    pltpu.sync_copy(cidx_hbm.at[wid], cidx_v)
    for j in range(_S):
        pltpu.sync_copy(kidx_hbm.at[j, wid], kidx_v.at[j])

    lane = lax.iota(jnp.int32, _LANES)

    def step(t, carry):
        descs = [pltpu.async_copy(in_hbm.at[cidx_v.at[t]], crow_v, sem)]
        for j in range(_S):
            descs.append(
                pltpu.async_copy(out_hbm.at[kidx_v.at[j, t]], orow_v.at[j],
                                 sem))
        for d in descs:
            d.wait()

        def group(g, carry):
            def dots(b, res):
                bb = g * _LANES + b
                c = [crow_v[bb, pl.ds(16 * k, 16)] for k in range(_DV)]
                cn = [-ck for ck in c]
                new = []
                for j in range(_S):
                    cc = c if j == 0 else cn
                    acc = cc[0] * orow_v[j, bb, pl.ds(0, 16)]
                    for k in range(1, _DV):
                        acc = acc + cc[k] * orow_v[j, bb, pl.ds(16 * k, 16)]
                    s = jnp.sum(acc)
                    new.append(jnp.where(lane == b, s, res[j]))
                return tuple(new)

            res = lax.fori_loop(
                0, _LANES, dots,
                tuple(jnp.zeros((_LANES,), jnp.float32) for _ in range(_S)))
            base = t * _BS + g * _LANES
            for j in range(_S):
                scores_v[j, pl.ds(base, _LANES)] = res[j]
            return carry

        return lax.fori_loop(0, _NG, group, carry)

    lax.fori_loop(0, _NSTEP, step, 0)

    pltpu.sync_copy(scores_v,
                    scores_hbm.at[:, pl.ds(wid * _CHUNK, _CHUNK)])


def _sc_scores():
    return pl.kernel(
        _sc_scores_body,
        out_type=jax.ShapeDtypeStruct((_S, _B), jnp.float32),
        mesh=plsc.VectorSubcoreMesh(
            core_axis_name="c", subcore_axis_name="s",
            num_cores=_NC, num_subcores=_NS),
        compiler_params=pltpu.CompilerParams(
            needs_layout_passes=False, use_tc_tiling_on_sc=False),
        scratch_types=[
            pltpu.VMEM((_NSTEP, _BS), jnp.int32),
            pltpu.VMEM((_S, _NSTEP, _BS), jnp.int32),
            pltpu.VMEM((_BS, _D), jnp.float32),
            pltpu.VMEM((_S, _BS, _D), jnp.float32),
            pltpu.VMEM((_S, _CHUNK), jnp.float32),
            pltpu.SemaphoreType.DMA,
        ],
    )


def _loss_body(s_ref, o_ref):
    i = pl.program_id(0)
    x = s_ref[...]
    ls = jnp.minimum(x, 0.0) - jnp.log1p(jnp.exp(-jnp.abs(x)))
    part = -jnp.sum(ls) / _B

    @pl.when(i == 0)
    def _():
        o_ref[0, 0] = 0.0

    o_ref[0, 0] += part


_COLS_PER_BLOCK = 2048


def _tc_loss(scores):
    out = pl.pallas_call(
        _loss_body,
        grid=(_B // _COLS_PER_BLOCK,),
        in_specs=[pl.BlockSpec((_S, _COLS_PER_BLOCK), lambda i: (0, i))],
        out_specs=pl.BlockSpec(memory_space=pltpu.SMEM),
        out_shape=jax.ShapeDtypeStruct((1, 1), jnp.float32),
    )(scores)
    return out[0, 0]


def kernel(center_idx, context_idx, neg_idx, in_emb, out_emb):
    cidx = center_idx.reshape(_NW, _NSTEP, _BS)
    kidx = jnp.concatenate([context_idx[None, :], neg_idx.T], axis=0)
    kidx = kidx.reshape(_S, _NW, _NSTEP, _BS)
    scores = _sc_scores()(cidx, kidx, in_emb, out_emb)
    return _tc_loss(scores)
